# baseline (speedup 1.0000x reference)
import jax
import jax.numpy as jnp
from jax.experimental import pallas as pl
from jax.experimental.pallas import tpu as pltpu

N, L, D1, D2, C = 50000, 1024, 256, 128, 2
TILE = 2000
NUM_TILES = N // TILE


def _tile_kernel(h_ref, w1_ref, b1_ref, wab_ref, bab_ref, wc_ref,
                 acc_ref, ms_ref):
    hh = jax.nn.relu(
        jnp.dot(h_ref[...].astype(jnp.bfloat16), w1_ref[...],
                preferred_element_type=jnp.float32)
        + b1_ref[...])
    ag = jnp.dot(hh.astype(jnp.bfloat16), wab_ref[...],
                 preferred_element_type=jnp.float32) + bab_ref[...]
    a = jnp.tanh(ag[:, :D2])
    g = jax.nn.sigmoid(ag[:, D2:])
    sc = jnp.dot(a * g, wc_ref[...],
                 preferred_element_type=jnp.float32)
    m_k = jnp.max(sc, axis=0, keepdims=True)
    p = jnp.exp(sc - m_k)
    acc_ref[...] = jnp.dot(p.T, hh,
                           preferred_element_type=jnp.float32)[None]
    ms = jnp.concatenate(
        [jnp.broadcast_to(m_k, (1, D2)),
         jnp.broadcast_to(jnp.sum(p, axis=0, keepdims=True), (1, D2))],
        axis=1)
    ms_ref[...] = ms[None]


def _combine_kernel(acc_ref, ms_ref, wcls_ref, bcls_ref,
                    logits_ref, prob_ref, yhat_ref):
    ms = ms_ref[:, 0, :]
    m_k = ms[:, 0:1]
    s_k = ms[:, D2:D2 + 1]
    m_g = jnp.max(m_k, axis=0, keepdims=True)
    w = jnp.exp(m_k - m_g)
    s = jnp.sum(w * s_k, axis=0, keepdims=True)
    acc = jnp.dot(w.T, acc_ref[:, 0, :],
                  preferred_element_type=jnp.float32)
    m_vec = acc / s
    logits = jnp.dot(m_vec, wcls_ref[...],
                     preferred_element_type=jnp.float32) + bcls_ref[...]
    logits_ref[...] = logits
    z = logits - jnp.max(logits, axis=1, keepdims=True)
    ez = jnp.exp(z)
    prob_ref[...] = ez / jnp.sum(ez, axis=1, keepdims=True)
    yhat_ref[...] = (logits[:, 1:2] > logits[:, 0:1]).astype(jnp.int32)


@jax.jit
def _run(h, W1, b1, Wab, bab, Wc, Wcls, bcls):
    full = lambda shape: pl.BlockSpec(shape, lambda i: (0, 0))
    acc, ms = pl.pallas_call(
        _tile_kernel,
        grid=(NUM_TILES,),
        in_specs=[
            pl.BlockSpec((TILE, L), lambda i: (i, 0)),
            full((L, D1)),
            full((1, D1)),
            full((D1, 2 * D2)),
            full((1, 2 * D2)),
            full((D2, 1)),
        ],
        out_specs=(
            pl.BlockSpec((1, 1, D1), lambda i: (i, 0, 0)),
            pl.BlockSpec((1, 1, 2 * D2), lambda i: (i, 0, 0)),
        ),
        out_shape=(
            jax.ShapeDtypeStruct((NUM_TILES, 1, D1), jnp.float32),
            jax.ShapeDtypeStruct((NUM_TILES, 1, 2 * D2), jnp.float32),
        ),
        compiler_params=pltpu.CompilerParams(
            dimension_semantics=("parallel",)),
    )(h, W1, b1, Wab, bab, Wc)
    return pl.pallas_call(
        _combine_kernel,
        out_shape=(
            jax.ShapeDtypeStruct((1, C), jnp.float32),
            jax.ShapeDtypeStruct((1, C), jnp.float32),
            jax.ShapeDtypeStruct((1, 1), jnp.int32),
        ),
    )(acc, ms, Wcls, bcls)


def kernel(h, label, W1, b1, Wa, ba, Wb, bb, Wc, bc, Wcls, bcls):
    del label, bc
    Wab = jnp.concatenate([Wa, Wb], axis=1).astype(jnp.bfloat16)
    bab = jnp.concatenate([ba, bb]).reshape(1, 2 * D2)
    logits, prob, yhat = _run(
        h, W1.astype(jnp.bfloat16), b1.reshape(1, D1), Wab, bab,
        Wc, Wcls, bcls.reshape(1, C))
    return (logits, prob, yhat)


# speedup vs baseline: 1.0317x; 1.0317x over previous
import jax
import jax.numpy as jnp
from jax.experimental import pallas as pl
from jax.experimental.pallas import tpu as pltpu

N, L, D1, D2, C = 50000, 1024, 256, 128, 2
TILE = 2000
NUM_TILES = N // (2 * TILE)
HALF_BLOCKS = NUM_TILES


def _attend(h_bf16, w1_ref, b1_ref, wab_ref, bab_ref, wc_ref):
    hh = jax.nn.relu(
        jnp.dot(h_bf16, w1_ref[...], preferred_element_type=jnp.float32)
        + b1_ref[...])
    ag = jnp.dot(hh.astype(jnp.bfloat16), wab_ref[...],
                 preferred_element_type=jnp.float32) + bab_ref[...]
    a = jnp.tanh(ag[:, :D2])
    g = jax.nn.sigmoid(ag[:, D2:])
    sc = jnp.dot(a * g, wc_ref[...],
                 preferred_element_type=jnp.float32)
    return hh, sc


def _clam_kernel(h1_ref, h2_ref, w1_ref, b1_ref, wab_ref, bab_ref,
                 wc_ref, wcls_ref, bcls_ref,
                 logits_ref, prob_ref, yhat_ref,
                 acc_ref, m_ref, s_ref):
    i = pl.program_id(0)

    @pl.when(i == 0)
    def _init():
        acc_ref[...] = jnp.zeros_like(acc_ref)
        m_ref[...] = jnp.full((1, 1), -jnp.inf, jnp.float32)
        s_ref[...] = jnp.zeros((1, 1), jnp.float32)

    hh1, sc1 = _attend(h1_ref[...].astype(jnp.bfloat16),
                       w1_ref, b1_ref, wab_ref, bab_ref, wc_ref)
    hh2, sc2 = _attend(h2_ref[...].astype(jnp.bfloat16),
                       w1_ref, b1_ref, wab_ref, bab_ref, wc_ref)

    m_old = m_ref[...]
    tile_max = jnp.maximum(jnp.max(sc1, axis=0, keepdims=True),
                           jnp.max(sc2, axis=0, keepdims=True))
    m_new = jnp.maximum(m_old, tile_max)
    alpha = jnp.exp(m_old - m_new)
    p1 = jnp.exp(sc1 - m_new)
    p2 = jnp.exp(sc2 - m_new)
    s_ref[...] = (s_ref[...] * alpha
                  + jnp.sum(p1, axis=0, keepdims=True)
                  + jnp.sum(p2, axis=0, keepdims=True))
    acc_ref[...] = (acc_ref[...] * alpha
                    + jnp.dot(p1.T, hh1, preferred_element_type=jnp.float32)
                    + jnp.dot(p2.T, hh2, preferred_element_type=jnp.float32))
    m_ref[...] = m_new

    @pl.when(i == NUM_TILES - 1)
    def _finish():
        m_vec = acc_ref[...] / s_ref[...]
        logits = jnp.dot(m_vec, wcls_ref[...],
                         preferred_element_type=jnp.float32) + bcls_ref[...]
        logits_ref[...] = logits
        z = logits - jnp.max(logits, axis=1, keepdims=True)
        ez = jnp.exp(z)
        prob_ref[...] = ez / jnp.sum(ez, axis=1, keepdims=True)
        yhat_ref[...] = (logits[:, 1:2] > logits[:, 0:1]).astype(jnp.int32)


@jax.jit
def _run(h, W1, b1, Wab, bab, Wc, Wcls, bcls):
    out_shapes = (
        jax.ShapeDtypeStruct((1, C), jnp.float32),
        jax.ShapeDtypeStruct((1, C), jnp.float32),
        jax.ShapeDtypeStruct((1, 1), jnp.int32),
    )
    full = lambda shape: pl.BlockSpec(shape, lambda i: (0, 0))
    return pl.pallas_call(
        _clam_kernel,
        grid=(NUM_TILES,),
        in_specs=[
            pl.BlockSpec((TILE, L), lambda i: (i, 0)),
            pl.BlockSpec((TILE, L), lambda i: (HALF_BLOCKS + i, 0)),
            full((L, D1)),
            full((1, D1)),
            full((D1, 2 * D2)),
            full((1, 2 * D2)),
            full((D2, 1)),
            full((D1, C)),
            full((1, C)),
        ],
        out_specs=(full((1, C)), full((1, C)), full((1, 1))),
        out_shape=out_shapes,
        scratch_shapes=[
            pltpu.VMEM((1, D1), jnp.float32),
            pltpu.VMEM((1, 1), jnp.float32),
            pltpu.VMEM((1, 1), jnp.float32),
        ],
    )(h, h, W1, b1, Wab, bab, Wc, Wcls, bcls)


def kernel(h, label, W1, b1, Wa, ba, Wb, bb, Wc, bc, Wcls, bcls):
    del label, bc
    Wab = jnp.concatenate([Wa, Wb], axis=1).astype(jnp.bfloat16)
    bab = jnp.concatenate([ba, bb]).reshape(1, 2 * D2)
    logits, prob, yhat = _run(
        h, W1.astype(jnp.bfloat16), b1.reshape(1, D1), Wab, bab,
        Wc, Wcls, bcls.reshape(1, C))
    return (logits, prob, yhat)


# speedup vs baseline: 1.0369x; 1.0050x over previous
import jax
import jax.numpy as jnp
from jax.experimental import pallas as pl
from jax.experimental.pallas import tpu as pltpu

N, L, D1, D2, C = 50000, 1024, 256, 128, 2
TILE = 5000
NUM_TILES = N // TILE


def _clam_kernel(h_ref, w1_ref, b1_ref, wab_ref, bab_ref,
                 wc_ref, wcls_ref, bcls_ref,
                 logits_ref, prob_ref, yhat_ref,
                 acc_ref, m_ref, s_ref):
    i = pl.program_id(0)

    @pl.when(i == 0)
    def _init():
        acc_ref[...] = jnp.zeros_like(acc_ref)
        m_ref[...] = jnp.full((1, 1), -jnp.inf, jnp.float32)
        s_ref[...] = jnp.zeros((1, 1), jnp.float32)

    hh = jax.nn.relu(
        jnp.dot(h_ref[...].astype(jnp.bfloat16), w1_ref[...],
                preferred_element_type=jnp.float32)
        + b1_ref[...])
    ag = jnp.dot(hh.astype(jnp.bfloat16), wab_ref[...],
                 preferred_element_type=jnp.float32) + bab_ref[...]
    a = jnp.tanh(ag[:, :D2])
    g = jax.nn.sigmoid(ag[:, D2:])
    sc = jnp.dot(a * g, wc_ref[...],
                 preferred_element_type=jnp.float32)

    m_old = m_ref[...]
    m_new = jnp.maximum(m_old, jnp.max(sc, axis=0, keepdims=True))
    alpha = jnp.exp(m_old - m_new)
    p = jnp.exp(sc - m_new)
    s_ref[...] = s_ref[...] * alpha + jnp.sum(p, axis=0, keepdims=True)
    acc_ref[...] = acc_ref[...] * alpha + jnp.dot(
        p.T, hh, preferred_element_type=jnp.float32)
    m_ref[...] = m_new

    @pl.when(i == NUM_TILES - 1)
    def _finish():
        m_vec = acc_ref[...] / s_ref[...]
        logits = jnp.dot(m_vec, wcls_ref[...],
                         preferred_element_type=jnp.float32) + bcls_ref[...]
        logits_ref[...] = logits
        z = logits - jnp.max(logits, axis=1, keepdims=True)
        ez = jnp.exp(z)
        prob_ref[...] = ez / jnp.sum(ez, axis=1, keepdims=True)
        yhat_ref[...] = (logits[:, 1:2] > logits[:, 0:1]).astype(jnp.int32)


@jax.jit
def _run(h, W1, b1, Wab, bab, Wc, Wcls, bcls):
    out_shapes = (
        jax.ShapeDtypeStruct((1, C), jnp.float32),
        jax.ShapeDtypeStruct((1, C), jnp.float32),
        jax.ShapeDtypeStruct((1, 1), jnp.int32),
    )
    full = lambda shape: pl.BlockSpec(shape, lambda i: (0, 0))
    return pl.pallas_call(
        _clam_kernel,
        grid=(NUM_TILES,),
        in_specs=[
            pl.BlockSpec((TILE, L), lambda i: (i, 0)),
            full((L, D1)),
            full((1, D1)),
            full((D1, 2 * D2)),
            full((1, 2 * D2)),
            full((D2, 1)),
            full((D1, C)),
            full((1, C)),
        ],
        out_specs=(full((1, C)), full((1, C)), full((1, 1))),
        out_shape=out_shapes,
        scratch_shapes=[
            pltpu.VMEM((1, D1), jnp.float32),
            pltpu.VMEM((1, 1), jnp.float32),
            pltpu.VMEM((1, 1), jnp.float32),
        ],
    )(h, W1, b1, Wab, bab, Wc, Wcls, bcls)


def kernel(h, label, W1, b1, Wa, ba, Wb, bb, Wc, bc, Wcls, bcls):
    del label, bc
    Wab = jnp.concatenate([Wa, Wb], axis=1).astype(jnp.bfloat16)
    bab = jnp.concatenate([ba, bb]).reshape(1, 2 * D2)
    logits, prob, yhat = _run(
        h, W1.astype(jnp.bfloat16), b1.reshape(1, D1), Wab, bab,
        Wc, Wcls, bcls.reshape(1, C))
    return (logits, prob, yhat)
